# baseline (speedup 1.0000x reference)
import jax
import jax.numpy as jnp
from jax.experimental import pallas as pl
from jax.experimental.pallas import tpu as pltpu
from jax.experimental.pallas import tpu_sc as plsc

E = 8
K = 2
S = 2048
H = 768
I = 3072
C = 768
EC = E * C
XG_ROWS = EC + 8
IBLK = 768
NI = I // IBLK
W_SC = 32
TS = 256

_HIGH = jax.lax.Precision.HIGHEST


def _router_body(x_ref, wr_ref, wc_ref, bc_ref,
                 slots_ref, w_ref, lbl_ref, z_ref, div_ref, cap_ref):
    x = x_ref[...]
    logits = jax.lax.dot_general(x, wr_ref[...], (((1,), (1,)), ((), ())))
    m = jnp.max(logits, axis=1, keepdims=True)
    ex = jnp.exp(logits - m)
    probs = ex / jnp.sum(ex, axis=1, keepdims=True)
    cap = jax.nn.sigmoid(
        jnp.sum(x * wc_ref[...], axis=1, keepdims=True) + bc_ref[0])

    idx = jax.lax.broadcasted_iota(jnp.int32, (S, E), 1)
    m1 = jnp.max(probs, axis=1, keepdims=True)
    i1 = jnp.min(jnp.where(probs == m1, idx, E), axis=1, keepdims=True)
    one1 = idx == i1
    masked = jnp.where(one1, -jnp.inf, probs)
    m2 = jnp.max(masked, axis=1, keepdims=True)
    i2 = jnp.min(jnp.where(masked == m2, idx, E), axis=1, keepdims=True)
    one2 = idx == i2

    t = jnp.exp(m2 - m1)
    w1 = cap / (1.0 + t)
    w2 = cap * t / (1.0 + t)

    cnt = one1.astype(jnp.float32) + one2.astype(jnp.float32)
    ir = jax.lax.broadcasted_iota(jnp.int32, (S, S), 0)
    ic = jax.lax.broadcasted_iota(jnp.int32, (S, S), 1)
    tri = (ir > ic).astype(jnp.float32)
    r_excl = jax.lax.dot_general(tri, cnt, (((1,), (0,)), ((), ())),
                                 precision=_HIGH)
    r1 = jnp.sum(jnp.where(one1, r_excl, 0.0), axis=1, keepdims=True)
    r2 = jnp.sum(jnp.where(one2, r_excl, 0.0), axis=1, keepdims=True)
    r1 = r1.astype(jnp.int32)
    r2 = r2.astype(jnp.int32)
    ok1 = r1 < C
    ok2 = r2 < C
    s1 = jnp.where(ok1, i1 * C + r1, EC)
    s2 = jnp.where(ok2, i2 * C + r2, EC)
    w1 = jnp.where(ok1, w1, 0.0)
    w2 = jnp.where(ok2, w2, 0.0)

    slots_ref[...] = jnp.concatenate([s1, s2], axis=1)
    w_ref[...] = jnp.concatenate([w1, w2], axis=1)

    counts = jnp.sum(cnt, axis=0, keepdims=True)
    mean_load = float(S * K) / E
    lbl_ref[...] = (jnp.sum((counts - mean_load) ** 2, axis=1, keepdims=True)
                    / float(E - 1) / (mean_load * mean_load))
    z_ref[...] = jnp.mean(
        jnp.log(jnp.sum(jnp.exp(probs), axis=1, keepdims=True)),
        keepdims=True)
    ep = jnp.mean(probs, axis=0, keepdims=True)
    div_ref[...] = -jnp.sum(ep * jnp.log(ep + 1e-8), axis=1, keepdims=True)
    cap_ref[...] = (jnp.mean(cap, keepdims=True) - 0.6) ** 2


def _run_router(xf, Wr, Wc, bc):
    scalar = jax.ShapeDtypeStruct((1, 1), jnp.float32)
    return pl.pallas_call(
        _router_body,
        in_specs=[
            pl.BlockSpec((S, H), lambda: (0, 0)),
            pl.BlockSpec((E, H), lambda: (0, 0)),
            pl.BlockSpec((1, H), lambda: (0, 0)),
            pl.BlockSpec(memory_space=pltpu.SMEM),
        ],
        out_shape=[
            jax.ShapeDtypeStruct((S, 2), jnp.int32),
            jax.ShapeDtypeStruct((S, 2), jnp.float32),
            scalar, scalar, scalar, scalar,
        ],
    )(xf, Wr, Wc, bc)


def _ffn_body(xg_ref, wg_ref, wu_ref, wd_ref, yg_ref):
    xb = xg_ref[...]
    g = jax.lax.dot_general(xb, wg_ref[0], (((1,), (1,)), ((), ())),
                            preferred_element_type=jnp.float32)
    u = jax.lax.dot_general(xb, wu_ref[0], (((1,), (1,)), ((), ())),
                            preferred_element_type=jnp.float32)
    h = (g * jax.nn.sigmoid(g) * u).astype(jnp.bfloat16)
    part = jax.lax.dot_general(h, wd_ref[0], (((1,), (1,)), ((), ())),
                               preferred_element_type=jnp.float32)

    @pl.when(pl.program_id(1) == 0)
    def _init():
        yg_ref[...] = part

    @pl.when(pl.program_id(1) != 0)
    def _acc():
        yg_ref[...] += part


def _run_ffn(xg, Wg, Wu, Wd):
    return pl.pallas_call(
        _ffn_body,
        grid=(E, NI),
        in_specs=[
            pl.BlockSpec((C, H), lambda e, i: (e, 0)),
            pl.BlockSpec((1, IBLK, H), lambda e, i: (e, i, 0)),
            pl.BlockSpec((1, IBLK, H), lambda e, i: (e, i, 0)),
            pl.BlockSpec((1, H, IBLK), lambda e, i: (e, 0, i)),
        ],
        out_specs=pl.BlockSpec((C, H), lambda e, i: (e, 0)),
        out_shape=jax.ShapeDtypeStruct((XG_ROWS, H), jnp.float32),
    )(xg, Wg, Wu, Wd)


def _vec_mesh():
    return plsc.VectorSubcoreMesh(core_axis_name="core",
                                  subcore_axis_name="subcore")


NSUB = 3
WSUB = H // NSUB
XW = H // 2
WSUB_W = XW // NSUB
NIDX = 2 * S * NSUB
IW = 128


@jax.jit
def _sc_dispatch(x_sub, s3):
    @pl.kernel(out_type=jax.ShapeDtypeStruct((XG_ROWS * NSUB, WSUB_W), jnp.int32),
               mesh=_vec_mesh())
    def k(x_hbm, s_hbm, o_hbm):
        def body(x_vmem, s_vmem):
            pltpu.sync_copy(x_vmem, o_hbm.at[s_vmem.at[0]])

        nxb = S * NSUB // IW
        pltpu.emit_pipeline(
            body,
            grid=(NIDX // IW,),
            in_specs=[
                pl.BlockSpec((IW, WSUB_W), index_map=lambda i: (i % nxb, 0)),
                pl.BlockSpec((1, IW), index_map=lambda i: (0, i)),
            ],
            out_specs=[],
            core_axis_name=("core", "subcore"),
            dimension_semantics=(pltpu.PARALLEL,),
        )(x_hbm, s_hbm)

    return k(x_sub, s3)


@jax.jit
def _sc_gather(yg_sub, s3):
    @pl.kernel(out_type=jax.ShapeDtypeStruct((NIDX, WSUB), jnp.float32),
               mesh=_vec_mesh())
    def k(y_hbm, s_hbm, o_hbm):
        def body(s_vmem, o_vmem):
            pltpu.sync_copy(y_hbm.at[s_vmem.at[0]], o_vmem)

        pltpu.emit_pipeline(
            body,
            grid=(NIDX // IW,),
            in_specs=[pl.BlockSpec((1, IW), index_map=lambda i: (0, i))],
            out_specs=[pl.BlockSpec((IW, WSUB), index_map=lambda i: (i, 0))],
            core_axis_name=("core", "subcore"),
            dimension_semantics=(pltpu.PARALLEL,),
        )(s_hbm, o_hbm)

    return k(yg_sub, s3)


def _combine_body(g1_ref, g2_ref, w_ref, o_ref):
    w = w_ref[...]
    o_ref[...] = w[:, 0:1] * g1_ref[...] + w[:, 1:2] * g2_ref[...]


def _run_combine(gath, wts):
    nt = S // TS
    return pl.pallas_call(
        _combine_body,
        grid=(nt,),
        in_specs=[
            pl.BlockSpec((TS, H), lambda i: (i, 0)),
            pl.BlockSpec((TS, H), lambda i: (i + nt, 0)),
            pl.BlockSpec((TS, 2), lambda i: (i, 0)),
        ],
        out_specs=pl.BlockSpec((TS, H), lambda i: (i, 0)),
        out_shape=jax.ShapeDtypeStruct((S, H), jnp.float32),
    )(gath, gath, wts)


def kernel(x, Wr, Wc, bc, Wg, Wu, Wd):
    B_, S_, H_ = x.shape
    xf = x.reshape(S, H)
    slots, wts, lbl, z, div, closs = _run_router(xf, Wr, Wc, bc)
    s_all = jnp.transpose(slots).reshape(2 * S, 1)
    s3 = (s_all * NSUB + jnp.arange(NSUB, dtype=jnp.int32)).reshape(1, NIDX)
    x16 = xf.astype(jnp.bfloat16)
    x_w = jax.lax.bitcast_convert_type(x16.reshape(S, XW, 2), jnp.int32)
    xg_w = _sc_dispatch(x_w.reshape(S * NSUB, WSUB_W), s3)
    xg16 = jax.lax.bitcast_convert_type(xg_w, jnp.bfloat16).reshape(XG_ROWS, H)
    yg = _run_ffn(xg16,
                  Wg.astype(jnp.bfloat16), Wu.astype(jnp.bfloat16),
                  Wd.astype(jnp.bfloat16))
    gath_sub = _sc_gather(yg.reshape(XG_ROWS * NSUB, WSUB), s3)
    out = _run_combine(gath_sub.reshape(2 * S, H), wts)
    return (out.reshape(B_, S_, H_),
            lbl.reshape(()), z.reshape(()), div.reshape(()), closs.reshape(()))


# speedup vs baseline: 9.8128x; 9.8128x over previous
import jax
import jax.numpy as jnp
from jax.experimental import pallas as pl
from jax.experimental.pallas import tpu as pltpu
from jax.experimental.pallas import tpu_sc as plsc

E = 8
K = 2
S = 2048
H = 768
I = 3072
C = 768
EC = E * C
XG_ROWS = EC + 8
IBLK = 768
NI = I // IBLK
W_SC = 32
TS = 256

_HIGH = jax.lax.Precision.HIGHEST


def _router_body(x_ref, wr_ref, wc_ref, bc_ref,
                 slots_ref, w_ref, lbl_ref, z_ref, div_ref, cap_ref):
    x = x_ref[...]
    logits = jax.lax.dot_general(x, wr_ref[...], (((1,), (1,)), ((), ())))
    m = jnp.max(logits, axis=1, keepdims=True)
    ex = jnp.exp(logits - m)
    probs = ex / jnp.sum(ex, axis=1, keepdims=True)
    cap = jax.nn.sigmoid(
        jnp.sum(x * wc_ref[...], axis=1, keepdims=True) + bc_ref[0])

    idx = jax.lax.broadcasted_iota(jnp.int32, (S, E), 1)
    m1 = jnp.max(probs, axis=1, keepdims=True)
    i1 = jnp.min(jnp.where(probs == m1, idx, E), axis=1, keepdims=True)
    one1 = idx == i1
    masked = jnp.where(one1, -jnp.inf, probs)
    m2 = jnp.max(masked, axis=1, keepdims=True)
    i2 = jnp.min(jnp.where(masked == m2, idx, E), axis=1, keepdims=True)
    one2 = idx == i2

    t = jnp.exp(m2 - m1)
    w1 = cap / (1.0 + t)
    w2 = cap * t / (1.0 + t)

    cnt = one1.astype(jnp.float32) + one2.astype(jnp.float32)
    ir = jax.lax.broadcasted_iota(jnp.int32, (S, S), 0)
    ic = jax.lax.broadcasted_iota(jnp.int32, (S, S), 1)
    tri = (ir > ic).astype(jnp.float32)
    r_excl = jax.lax.dot_general(tri, cnt, (((1,), (0,)), ((), ())),
                                 precision=_HIGH)
    r1 = jnp.sum(jnp.where(one1, r_excl, 0.0), axis=1, keepdims=True)
    r2 = jnp.sum(jnp.where(one2, r_excl, 0.0), axis=1, keepdims=True)
    r1 = r1.astype(jnp.int32)
    r2 = r2.astype(jnp.int32)
    ok1 = r1 < C
    ok2 = r2 < C
    s1 = jnp.where(ok1, i1 * C + r1, EC)
    s2 = jnp.where(ok2, i2 * C + r2, EC)
    w1 = jnp.where(ok1, w1, 0.0)
    w2 = jnp.where(ok2, w2, 0.0)

    slots_ref[...] = jnp.concatenate([s1, s2], axis=1)
    w_ref[...] = jnp.concatenate([w1, w2], axis=1)

    counts = jnp.sum(cnt, axis=0, keepdims=True)
    mean_load = float(S * K) / E
    lbl_ref[...] = (jnp.sum((counts - mean_load) ** 2, axis=1, keepdims=True)
                    / float(E - 1) / (mean_load * mean_load))
    z_ref[...] = jnp.mean(
        jnp.log(jnp.sum(jnp.exp(probs), axis=1, keepdims=True)),
        keepdims=True)
    ep = jnp.mean(probs, axis=0, keepdims=True)
    div_ref[...] = -jnp.sum(ep * jnp.log(ep + 1e-8), axis=1, keepdims=True)
    cap_ref[...] = (jnp.mean(cap, keepdims=True) - 0.6) ** 2


def _run_router(xf, Wr, Wc, bc):
    scalar = jax.ShapeDtypeStruct((1, 1), jnp.float32)
    return pl.pallas_call(
        _router_body,
        in_specs=[
            pl.BlockSpec((S, H), lambda: (0, 0)),
            pl.BlockSpec((E, H), lambda: (0, 0)),
            pl.BlockSpec((1, H), lambda: (0, 0)),
            pl.BlockSpec(memory_space=pltpu.SMEM),
        ],
        out_shape=[
            jax.ShapeDtypeStruct((S, 2), jnp.int32),
            jax.ShapeDtypeStruct((S, 2), jnp.float32),
            scalar, scalar, scalar, scalar,
        ],
    )(xf, Wr, Wc, bc)


def _ffn_body(xg_ref, wg_ref, wu_ref, wd_ref, yg_ref):
    xb = xg_ref[...].astype(jnp.bfloat16)
    g = jax.lax.dot_general(xb, wg_ref[0], (((1,), (1,)), ((), ())),
                            preferred_element_type=jnp.float32)
    u = jax.lax.dot_general(xb, wu_ref[0], (((1,), (1,)), ((), ())),
                            preferred_element_type=jnp.float32)
    h = (g * jax.nn.sigmoid(g) * u).astype(jnp.bfloat16)
    part = jax.lax.dot_general(h, wd_ref[0], (((1,), (1,)), ((), ())),
                               preferred_element_type=jnp.float32)

    @pl.when(pl.program_id(1) == 0)
    def _init():
        yg_ref[...] = part

    @pl.when(pl.program_id(1) != 0)
    def _acc():
        yg_ref[...] += part


def _run_ffn(xg, Wg, Wu, Wd):
    return pl.pallas_call(
        _ffn_body,
        grid=(E, NI),
        in_specs=[
            pl.BlockSpec((C, H), lambda e, i: (e, 0)),
            pl.BlockSpec((1, IBLK, H), lambda e, i: (e, i, 0)),
            pl.BlockSpec((1, IBLK, H), lambda e, i: (e, i, 0)),
            pl.BlockSpec((1, H, IBLK), lambda e, i: (e, 0, i)),
        ],
        out_specs=pl.BlockSpec((C, H), lambda e, i: (e, 0)),
        out_shape=jax.ShapeDtypeStruct((XG_ROWS, H), jnp.float32),
    )(xg, Wg, Wu, Wd)


def _vec_mesh():
    return plsc.VectorSubcoreMesh(core_axis_name="core",
                                  subcore_axis_name="subcore")


NSUB = 3
WSUB = H // NSUB
XW = H // 2
WSUB_W = XW // NSUB
NIDX = 2 * S * NSUB
IW = 128


@jax.jit
def _sc_dispatch(x_sub, s3):
    @pl.kernel(out_type=jax.ShapeDtypeStruct((XG_ROWS * NSUB, WSUB), jnp.float32),
               mesh=_vec_mesh())
    def k(x_hbm, s_hbm, o_hbm):
        def body(x_vmem, s_vmem):
            pltpu.sync_copy(x_vmem, o_hbm.at[s_vmem.at[0]])

        nxb = S * NSUB // IW
        pltpu.emit_pipeline(
            body,
            grid=(NIDX // IW,),
            in_specs=[
                pl.BlockSpec((IW, WSUB), index_map=lambda i: (i % nxb, 0)),
                pl.BlockSpec((1, IW), index_map=lambda i: (0, i)),
            ],
            out_specs=[],
            core_axis_name=("core", "subcore"),
            dimension_semantics=(pltpu.PARALLEL,),
        )(x_hbm, s_hbm)

    return k(x_sub, s3)


@jax.jit
def _sc_gather(yg_sub, s3):
    @pl.kernel(out_type=jax.ShapeDtypeStruct((NIDX, WSUB), jnp.float32),
               mesh=_vec_mesh())
    def k(y_hbm, s_hbm, o_hbm):
        def body(s_vmem, o_vmem):
            pltpu.sync_copy(y_hbm.at[s_vmem.at[0]], o_vmem)

        pltpu.emit_pipeline(
            body,
            grid=(NIDX // IW,),
            in_specs=[pl.BlockSpec((1, IW), index_map=lambda i: (0, i))],
            out_specs=[pl.BlockSpec((IW, WSUB), index_map=lambda i: (i, 0))],
            core_axis_name=("core", "subcore"),
            dimension_semantics=(pltpu.PARALLEL,),
        )(s_hbm, o_hbm)

    return k(yg_sub, s3)


def _combine_body(g1_ref, g2_ref, w_ref, o_ref):
    w = w_ref[...]
    o_ref[...] = w[:, 0:1] * g1_ref[...] + w[:, 1:2] * g2_ref[...]


def _run_combine(gath, wts):
    nt = S // TS
    return pl.pallas_call(
        _combine_body,
        grid=(nt,),
        in_specs=[
            pl.BlockSpec((TS, H), lambda i: (i, 0)),
            pl.BlockSpec((TS, H), lambda i: (i + nt, 0)),
            pl.BlockSpec((TS, 2), lambda i: (i, 0)),
        ],
        out_specs=pl.BlockSpec((TS, H), lambda i: (i, 0)),
        out_shape=jax.ShapeDtypeStruct((S, H), jnp.float32),
    )(gath, gath, wts)


def kernel(x, Wr, Wc, bc, Wg, Wu, Wd):
    B_, S_, H_ = x.shape
    xf = x.reshape(S, H)
    slots, wts, lbl, z, div, closs = _run_router(xf, Wr, Wc, bc)
    s_all = jnp.transpose(slots).reshape(2 * S, 1)
    s3 = (s_all * NSUB + jnp.arange(NSUB, dtype=jnp.int32)).reshape(1, NIDX)
    xg_sub = _sc_dispatch(xf.reshape(S * NSUB, WSUB), s3)
    yg = _run_ffn(xg_sub.reshape(XG_ROWS, H),
                  Wg.astype(jnp.bfloat16), Wu.astype(jnp.bfloat16),
                  Wd.astype(jnp.bfloat16))
    gath_sub = _sc_gather(yg.reshape(XG_ROWS * NSUB, WSUB), s3)
    out = _run_combine(gath_sub.reshape(2 * S, H), wts)
    return (out.reshape(B_, S_, H_),
            lbl.reshape(()), z.reshape(()), div.reshape(()), closs.reshape(()))


# speedup vs baseline: 13.4111x; 1.3667x over previous
import jax
import jax.numpy as jnp
from jax.experimental import pallas as pl
from jax.experimental.pallas import tpu as pltpu
from jax.experimental.pallas import tpu_sc as plsc

E = 8
K = 2
S = 2048
H = 768
I = 3072
C = 768
EC = E * C
XG_ROWS = EC + 8
IBLK = 768
NI = I // IBLK
W_SC = 32
TS = 256

_HIGH = jax.lax.Precision.HIGHEST


def _router_body(x_ref, wr_ref, wc_ref, bc_ref,
                 slots_ref, w_ref, lbl_ref, z_ref, div_ref, cap_ref):
    x = x_ref[...]
    logits = jax.lax.dot_general(x, wr_ref[...], (((1,), (1,)), ((), ())))
    m = jnp.max(logits, axis=1, keepdims=True)
    ex = jnp.exp(logits - m)
    probs = ex / jnp.sum(ex, axis=1, keepdims=True)
    cap = jax.nn.sigmoid(
        jnp.sum(x * wc_ref[...], axis=1, keepdims=True) + bc_ref[0])

    idx = jax.lax.broadcasted_iota(jnp.int32, (S, E), 1)
    m1 = jnp.max(probs, axis=1, keepdims=True)
    i1 = jnp.min(jnp.where(probs == m1, idx, E), axis=1, keepdims=True)
    one1 = idx == i1
    masked = jnp.where(one1, -jnp.inf, probs)
    m2 = jnp.max(masked, axis=1, keepdims=True)
    i2 = jnp.min(jnp.where(masked == m2, idx, E), axis=1, keepdims=True)
    one2 = idx == i2

    t = jnp.exp(m2 - m1)
    w1 = cap / (1.0 + t)
    w2 = cap * t / (1.0 + t)

    cnt = one1.astype(jnp.float32) + one2.astype(jnp.float32)
    ir = jax.lax.broadcasted_iota(jnp.int32, (S, S), 0)
    ic = jax.lax.broadcasted_iota(jnp.int32, (S, S), 1)
    tri = (ir > ic).astype(jnp.float32)
    r_excl = jax.lax.dot_general(tri, cnt, (((1,), (0,)), ((), ())),
                                 precision=_HIGH)
    r1 = jnp.sum(jnp.where(one1, r_excl, 0.0), axis=1, keepdims=True)
    r2 = jnp.sum(jnp.where(one2, r_excl, 0.0), axis=1, keepdims=True)
    r1 = r1.astype(jnp.int32)
    r2 = r2.astype(jnp.int32)
    ok1 = r1 < C
    ok2 = r2 < C
    s1 = jnp.where(ok1, i1 * C + r1, EC)
    s2 = jnp.where(ok2, i2 * C + r2, EC)
    w1 = jnp.where(ok1, w1, 0.0)
    w2 = jnp.where(ok2, w2, 0.0)

    slots_ref[...] = jnp.concatenate([s1, s2], axis=1)
    w_ref[...] = jnp.concatenate([w1, w2], axis=1)

    counts = jnp.sum(cnt, axis=0, keepdims=True)
    mean_load = float(S * K) / E
    lbl_ref[...] = (jnp.sum((counts - mean_load) ** 2, axis=1, keepdims=True)
                    / float(E - 1) / (mean_load * mean_load))
    z_ref[...] = jnp.mean(
        jnp.log(jnp.sum(jnp.exp(probs), axis=1, keepdims=True)),
        keepdims=True)
    ep = jnp.mean(probs, axis=0, keepdims=True)
    div_ref[...] = -jnp.sum(ep * jnp.log(ep + 1e-8), axis=1, keepdims=True)
    cap_ref[...] = (jnp.mean(cap, keepdims=True) - 0.6) ** 2


def _run_router(xf, Wr, Wc, bc):
    scalar = jax.ShapeDtypeStruct((1, 1), jnp.float32)
    return pl.pallas_call(
        _router_body,
        in_specs=[
            pl.BlockSpec((S, H), lambda: (0, 0)),
            pl.BlockSpec((E, H), lambda: (0, 0)),
            pl.BlockSpec((1, H), lambda: (0, 0)),
            pl.BlockSpec(memory_space=pltpu.SMEM),
        ],
        out_shape=[
            jax.ShapeDtypeStruct((S, 2), jnp.int32),
            jax.ShapeDtypeStruct((S, 2), jnp.float32),
            scalar, scalar, scalar, scalar,
        ],
    )(xf, Wr, Wc, bc)


def _ffn_body(xg_ref, wg_ref, wu_ref, wd_ref, yg_ref):
    xb = xg_ref[...].astype(jnp.bfloat16)
    g = jax.lax.dot_general(xb, wg_ref[0].astype(jnp.bfloat16),
                            (((1,), (1,)), ((), ())),
                            preferred_element_type=jnp.float32)
    u = jax.lax.dot_general(xb, wu_ref[0].astype(jnp.bfloat16),
                            (((1,), (1,)), ((), ())),
                            preferred_element_type=jnp.float32)
    h = (g * jax.nn.sigmoid(g) * u).astype(jnp.bfloat16)
    part = jax.lax.dot_general(h, wd_ref[0].astype(jnp.bfloat16),
                               (((1,), (1,)), ((), ())),
                               preferred_element_type=jnp.float32)

    @pl.when(pl.program_id(1) == 0)
    def _init():
        yg_ref[...] = part

    @pl.when(pl.program_id(1) != 0)
    def _acc():
        yg_ref[...] += part


def _run_ffn(xg, Wg, Wu, Wd):
    return pl.pallas_call(
        _ffn_body,
        grid=(E, NI),
        in_specs=[
            pl.BlockSpec((C, H), lambda e, i: (e, 0)),
            pl.BlockSpec((1, IBLK, H), lambda e, i: (e, i, 0)),
            pl.BlockSpec((1, IBLK, H), lambda e, i: (e, i, 0)),
            pl.BlockSpec((1, H, IBLK), lambda e, i: (e, 0, i)),
        ],
        out_specs=pl.BlockSpec((C, H), lambda e, i: (e, 0)),
        out_shape=jax.ShapeDtypeStruct((XG_ROWS, H), jnp.float32),
    )(xg, Wg, Wu, Wd)


def _vec_mesh():
    return plsc.VectorSubcoreMesh(core_axis_name="core",
                                  subcore_axis_name="subcore")


NSUB = 3
WSUB = H // NSUB
XW = H // 2
WSUB_W = XW // NSUB
NIDX = 2 * S * NSUB
IW = 128


@jax.jit
def _sc_dispatch(x_sub, s3):
    @pl.kernel(out_type=jax.ShapeDtypeStruct((XG_ROWS * NSUB, WSUB), jnp.float32),
               mesh=_vec_mesh())
    def k(x_hbm, s_hbm, o_hbm):
        def body(x_vmem, s_vmem):
            pltpu.sync_copy(x_vmem, o_hbm.at[s_vmem.at[0]])

        nxb = S * NSUB // IW
        pltpu.emit_pipeline(
            body,
            grid=(NIDX // IW,),
            in_specs=[
                pl.BlockSpec((IW, WSUB), index_map=lambda i: (i % nxb, 0)),
                pl.BlockSpec((1, IW), index_map=lambda i: (0, i)),
            ],
            out_specs=[],
            core_axis_name=("core", "subcore"),
            dimension_semantics=(pltpu.PARALLEL,),
        )(x_hbm, s_hbm)

    return k(x_sub, s3)


@jax.jit
def _sc_gather(yg_sub, s3):
    @pl.kernel(out_type=jax.ShapeDtypeStruct((NIDX, WSUB), jnp.float32),
               mesh=_vec_mesh())
    def k(y_hbm, s_hbm, o_hbm):
        def body(s_vmem, o_vmem):
            pltpu.sync_copy(y_hbm.at[s_vmem.at[0]], o_vmem)

        pltpu.emit_pipeline(
            body,
            grid=(NIDX // IW,),
            in_specs=[pl.BlockSpec((1, IW), index_map=lambda i: (0, i))],
            out_specs=[pl.BlockSpec((IW, WSUB), index_map=lambda i: (i, 0))],
            core_axis_name=("core", "subcore"),
            dimension_semantics=(pltpu.PARALLEL,),
        )(s_hbm, o_hbm)

    return k(yg_sub, s3)


def _combine_body(g1_ref, g2_ref, w_ref, o_ref):
    w = w_ref[...]
    o_ref[...] = w[:, 0:1] * g1_ref[...] + w[:, 1:2] * g2_ref[...]


def _run_combine(gath, wts):
    nt = S // TS
    return pl.pallas_call(
        _combine_body,
        grid=(nt,),
        in_specs=[
            pl.BlockSpec((TS, H), lambda i: (i, 0)),
            pl.BlockSpec((TS, H), lambda i: (i + nt, 0)),
            pl.BlockSpec((TS, 2), lambda i: (i, 0)),
        ],
        out_specs=pl.BlockSpec((TS, H), lambda i: (i, 0)),
        out_shape=jax.ShapeDtypeStruct((S, H), jnp.float32),
    )(gath, gath, wts)


def kernel(x, Wr, Wc, bc, Wg, Wu, Wd):
    B_, S_, H_ = x.shape
    xf = x.reshape(S, H)
    slots, wts, lbl, z, div, closs = _run_router(xf, Wr, Wc, bc)
    s_all = jnp.transpose(slots).reshape(2 * S, 1)
    s3 = (s_all * NSUB + jnp.arange(NSUB, dtype=jnp.int32)).reshape(1, NIDX)
    xg_sub = _sc_dispatch(xf.reshape(S * NSUB, WSUB), s3)
    yg = _run_ffn(xg_sub.reshape(XG_ROWS, H), Wg, Wu, Wd)
    gath_sub = _sc_gather(yg.reshape(XG_ROWS * NSUB, WSUB), s3)
    out = _run_combine(gath_sub.reshape(2 * S, H), wts)
    return (out.reshape(B_, S_, H_),
            lbl.reshape(()), z.reshape(()), div.reshape(()), closs.reshape(()))


# speedup vs baseline: 16.2971x; 1.2152x over previous
import jax
import jax.numpy as jnp
from jax.experimental import pallas as pl
from jax.experimental.pallas import tpu as pltpu
from jax.experimental.pallas import tpu_sc as plsc

E = 8
K = 2
S = 2048
H = 768
I = 3072
C = 768
EC = E * C
XG_ROWS = EC + 8
IBLK = 768
NI = I // IBLK
TS = 256

NP = 2
WP = H // NP
NIDX = K * NP * S
IW = 128

_HIGH = jax.lax.Precision.HIGHEST


def _router_body(x_ref, wr_ref, wc_ref, bc_ref,
                 slots_ref, w_ref, xp_ref, lbl_ref, z_ref, div_ref, cap_ref):
    x = x_ref[...]
    xp_ref[0] = x[:, :WP]
    xp_ref[1] = x[:, WP:]
    logits = jax.lax.dot_general(x, wr_ref[...], (((1,), (1,)), ((), ())))
    m = jnp.max(logits, axis=1, keepdims=True)
    ex = jnp.exp(logits - m)
    probs = ex / jnp.sum(ex, axis=1, keepdims=True)
    cap = jax.nn.sigmoid(
        jnp.sum(x * wc_ref[...], axis=1, keepdims=True) + bc_ref[0])

    idx = jax.lax.broadcasted_iota(jnp.int32, (S, E), 1)
    m1 = jnp.max(probs, axis=1, keepdims=True)
    i1 = jnp.min(jnp.where(probs == m1, idx, E), axis=1, keepdims=True)
    one1 = idx == i1
    masked = jnp.where(one1, -jnp.inf, probs)
    m2 = jnp.max(masked, axis=1, keepdims=True)
    i2 = jnp.min(jnp.where(masked == m2, idx, E), axis=1, keepdims=True)
    one2 = idx == i2

    t = jnp.exp(m2 - m1)
    w1 = cap / (1.0 + t)
    w2 = cap * t / (1.0 + t)

    cnt = one1.astype(jnp.float32) + one2.astype(jnp.float32)
    ir = jax.lax.broadcasted_iota(jnp.int32, (S, S), 0)
    ic = jax.lax.broadcasted_iota(jnp.int32, (S, S), 1)
    tri = (ir > ic).astype(jnp.float32)
    r_excl = jax.lax.dot_general(tri, cnt, (((1,), (0,)), ((), ())),
                                 precision=_HIGH)
    r1 = jnp.sum(jnp.where(one1, r_excl, 0.0), axis=1, keepdims=True)
    r2 = jnp.sum(jnp.where(one2, r_excl, 0.0), axis=1, keepdims=True)
    r1 = r1.astype(jnp.int32)
    r2 = r2.astype(jnp.int32)
    ok1 = r1 < C
    ok2 = r2 < C
    s1 = jnp.where(ok1, i1 * C + r1, EC)
    s2 = jnp.where(ok2, i2 * C + r2, EC)
    w1 = jnp.where(ok1, w1, 0.0)
    w2 = jnp.where(ok2, w2, 0.0)

    slots_ref[...] = jnp.concatenate([s1, s2], axis=1)
    w_ref[...] = jnp.concatenate([w1, w2], axis=1)

    counts = jnp.sum(cnt, axis=0, keepdims=True)
    mean_load = float(S * K) / E
    lbl_ref[...] = (jnp.sum((counts - mean_load) ** 2, axis=1, keepdims=True)
                    / float(E - 1) / (mean_load * mean_load))
    z_ref[...] = jnp.mean(
        jnp.log(jnp.sum(jnp.exp(probs), axis=1, keepdims=True)),
        keepdims=True)
    ep = jnp.mean(probs, axis=0, keepdims=True)
    div_ref[...] = -jnp.sum(ep * jnp.log(ep + 1e-8), axis=1, keepdims=True)
    cap_ref[...] = (jnp.mean(cap, keepdims=True) - 0.6) ** 2


def _run_router(xf, Wr, Wc, bc):
    scalar = jax.ShapeDtypeStruct((1, 1), jnp.float32)
    return pl.pallas_call(
        _router_body,
        in_specs=[
            pl.BlockSpec((S, H), lambda: (0, 0)),
            pl.BlockSpec((E, H), lambda: (0, 0)),
            pl.BlockSpec((1, H), lambda: (0, 0)),
            pl.BlockSpec(memory_space=pltpu.SMEM),
        ],
        out_shape=[
            jax.ShapeDtypeStruct((S, 2), jnp.int32),
            jax.ShapeDtypeStruct((S, 2), jnp.float32),
            jax.ShapeDtypeStruct((NP, S, WP), jnp.float32),
            scalar, scalar, scalar, scalar,
        ],
    )(xf, Wr, Wc, bc)


def _ffn_body(xg_ref, wg_ref, wu_ref, wd_ref, yg_ref):
    x0 = xg_ref[0].astype(jnp.bfloat16)
    x1 = xg_ref[1].astype(jnp.bfloat16)
    wg = wg_ref[0].astype(jnp.bfloat16)
    wu = wu_ref[0].astype(jnp.bfloat16)
    wd = wd_ref[0].astype(jnp.bfloat16)
    dims = (((1,), (1,)), ((), ()))
    g = (jax.lax.dot_general(x0, wg[:, :WP], dims,
                             preferred_element_type=jnp.float32)
         + jax.lax.dot_general(x1, wg[:, WP:], dims,
                               preferred_element_type=jnp.float32))
    u = (jax.lax.dot_general(x0, wu[:, :WP], dims,
                             preferred_element_type=jnp.float32)
         + jax.lax.dot_general(x1, wu[:, WP:], dims,
                               preferred_element_type=jnp.float32))
    h = (g * jax.nn.sigmoid(g) * u).astype(jnp.bfloat16)
    part = jax.lax.dot_general(h, wd, dims,
                               preferred_element_type=jnp.float32)

    @pl.when(pl.program_id(1) == 0)
    def _init():
        yg_ref[0] = part[:, :WP]
        yg_ref[1] = part[:, WP:]

    @pl.when(pl.program_id(1) != 0)
    def _acc():
        yg_ref[0] += part[:, :WP]
        yg_ref[1] += part[:, WP:]


def _run_ffn(xg_p, Wg, Wu, Wd):
    return pl.pallas_call(
        _ffn_body,
        grid=(E, NI),
        in_specs=[
            pl.BlockSpec((NP, C, WP), lambda e, i: (0, e, 0)),
            pl.BlockSpec((1, IBLK, H), lambda e, i: (e, i, 0)),
            pl.BlockSpec((1, IBLK, H), lambda e, i: (e, i, 0)),
            pl.BlockSpec((1, H, IBLK), lambda e, i: (e, 0, i)),
        ],
        out_specs=pl.BlockSpec((NP, C, WP), lambda e, i: (0, e, 0)),
        out_shape=jax.ShapeDtypeStruct((NP, XG_ROWS, WP), jnp.float32),
    )(xg_p, Wg, Wu, Wd)


def _vec_mesh():
    return plsc.VectorSubcoreMesh(core_axis_name="core",
                                  subcore_axis_name="subcore")


@jax.jit
def _sc_dispatch(x_p, sp):
    @pl.kernel(out_type=jax.ShapeDtypeStruct((NP * XG_ROWS, WP), jnp.float32),
               mesh=_vec_mesh())
    def k(x_hbm, s_hbm, o_hbm):
        def body(x_vmem, s_vmem):
            pltpu.sync_copy(x_vmem, o_hbm.at[s_vmem.at[0]])

        nxb = NP * S // IW
        pltpu.emit_pipeline(
            body,
            grid=(NIDX // IW,),
            in_specs=[
                pl.BlockSpec((IW, WP), index_map=lambda i: (i % nxb, 0)),
                pl.BlockSpec((1, IW), index_map=lambda i: (0, i)),
            ],
            out_specs=[],
            core_axis_name=("core", "subcore"),
            dimension_semantics=(pltpu.PARALLEL,),
        )(x_hbm, s_hbm)

    return k(x_p, sp)


@jax.jit
def _sc_gather(yg_p, sp):
    @pl.kernel(out_type=jax.ShapeDtypeStruct((NIDX, WP), jnp.float32),
               mesh=_vec_mesh())
    def k(y_hbm, s_hbm, o_hbm):
        def body(s_vmem, o_vmem):
            pltpu.sync_copy(y_hbm.at[s_vmem.at[0]], o_vmem)

        pltpu.emit_pipeline(
            body,
            grid=(NIDX // IW,),
            in_specs=[pl.BlockSpec((1, IW), index_map=lambda i: (0, i))],
            out_specs=[pl.BlockSpec((IW, WP), index_map=lambda i: (i, 0))],
            core_axis_name=("core", "subcore"),
            dimension_semantics=(pltpu.PARALLEL,),
        )(s_hbm, o_hbm)

    return k(yg_p, sp)


def _combine_body(g1_ref, g2_ref, w_ref, o_ref):
    w = w_ref[...]
    w1 = w[:, 0:1]
    w2 = w[:, 1:2]
    o_ref[...] = jnp.concatenate(
        [w1 * g1_ref[0, 0] + w2 * g2_ref[0, 0],
         w1 * g1_ref[0, 1] + w2 * g2_ref[0, 1]], axis=1)


def _run_combine(gath4, wts):
    return pl.pallas_call(
        _combine_body,
        grid=(S // TS,),
        in_specs=[
            pl.BlockSpec((1, NP, TS, WP), lambda i: (0, 0, i, 0)),
            pl.BlockSpec((1, NP, TS, WP), lambda i: (1, 0, i, 0)),
            pl.BlockSpec((TS, 2), lambda i: (i, 0)),
        ],
        out_specs=pl.BlockSpec((TS, H), lambda i: (i, 0)),
        out_shape=jax.ShapeDtypeStruct((S, H), jnp.float32),
    )(gath4, gath4, wts)


def kernel(x, Wr, Wc, bc, Wg, Wu, Wd):
    B_, S_, H_ = x.shape
    xf = x.reshape(S, H)
    slots, wts, x_p, lbl, z, div, closs = _run_router(xf, Wr, Wc, bc)
    plane_off = jnp.arange(NP, dtype=jnp.int32)[None, :, None] * XG_ROWS
    s_kt = jnp.transpose(slots)
    sp = (plane_off + s_kt[:, None, :]).reshape(1, NIDX)
    xg_p = _sc_dispatch(x_p.reshape(NP * S, WP), sp)
    yg_p = _run_ffn(xg_p.reshape(NP, XG_ROWS, WP), Wg, Wu, Wd)
    gath = _sc_gather(yg_p.reshape(NP * XG_ROWS, WP), sp)
    out = _run_combine(gath.reshape(K, NP, S, WP), wts)
    return (out.reshape(B_, S_, H_),
            lbl.reshape(()), z.reshape(()), div.reshape(()), closs.reshape(()))


# speedup vs baseline: 16.3087x; 1.0007x over previous
import jax
import jax.numpy as jnp
from jax.experimental import pallas as pl
from jax.experimental.pallas import tpu as pltpu
from jax.experimental.pallas import tpu_sc as plsc

E = 8
K = 2
S = 2048
H = 768
I = 3072
C = 768
EC = E * C
XG_ROWS = EC + 8
IBLK = 768
NI = I // IBLK
TS = 256

NP = 2
WP = H // NP
NIDX = K * NP * S
IW = 128

_HIGH = jax.lax.Precision.HIGHEST


def _router_body(x_ref, wr_ref, wc_ref, bc_ref,
                 slots_ref, w_ref, xp_ref, lbl_ref, z_ref, div_ref, cap_ref):
    x = x_ref[...]
    xp_ref[0] = x[:, :WP]
    xp_ref[1] = x[:, WP:]
    logits = jax.lax.dot_general(x, wr_ref[...], (((1,), (1,)), ((), ())))
    m = jnp.max(logits, axis=1, keepdims=True)
    ex = jnp.exp(logits - m)
    probs = ex / jnp.sum(ex, axis=1, keepdims=True)
    cap = jax.nn.sigmoid(
        jnp.sum(x * wc_ref[...], axis=1, keepdims=True) + bc_ref[0])

    idx = jax.lax.broadcasted_iota(jnp.int32, (S, E), 1)
    m1 = jnp.max(probs, axis=1, keepdims=True)
    i1 = jnp.min(jnp.where(probs == m1, idx, E), axis=1, keepdims=True)
    one1 = idx == i1
    masked = jnp.where(one1, -jnp.inf, probs)
    m2 = jnp.max(masked, axis=1, keepdims=True)
    i2 = jnp.min(jnp.where(masked == m2, idx, E), axis=1, keepdims=True)
    one2 = idx == i2

    t = jnp.exp(m2 - m1)
    w1 = cap / (1.0 + t)
    w2 = cap * t / (1.0 + t)

    cnt = one1.astype(jnp.float32) + one2.astype(jnp.float32)
    ir = jax.lax.broadcasted_iota(jnp.int32, (S, S), 0)
    ic = jax.lax.broadcasted_iota(jnp.int32, (S, S), 1)
    tri = (ir > ic).astype(jnp.float32)
    r_excl = jax.lax.dot_general(tri, cnt, (((1,), (0,)), ((), ())),
                                 precision=_HIGH)
    r1 = jnp.sum(jnp.where(one1, r_excl, 0.0), axis=1, keepdims=True)
    r2 = jnp.sum(jnp.where(one2, r_excl, 0.0), axis=1, keepdims=True)
    r1 = r1.astype(jnp.int32)
    r2 = r2.astype(jnp.int32)
    ok1 = r1 < C
    ok2 = r2 < C
    s1 = jnp.where(ok1, i1 * C + r1, EC)
    s2 = jnp.where(ok2, i2 * C + r2, EC)
    w1 = jnp.where(ok1, w1, 0.0)
    w2 = jnp.where(ok2, w2, 0.0)

    slots_ref[...] = jnp.concatenate([s1, s2], axis=1)
    w_ref[...] = jnp.concatenate([w1, w2], axis=1)

    counts = jnp.sum(cnt, axis=0, keepdims=True)
    mean_load = float(S * K) / E
    lbl_ref[...] = (jnp.sum((counts - mean_load) ** 2, axis=1, keepdims=True)
                    / float(E - 1) / (mean_load * mean_load))
    z_ref[...] = jnp.mean(
        jnp.log(jnp.sum(jnp.exp(probs), axis=1, keepdims=True)),
        keepdims=True)
    ep = jnp.mean(probs, axis=0, keepdims=True)
    div_ref[...] = -jnp.sum(ep * jnp.log(ep + 1e-8), axis=1, keepdims=True)
    cap_ref[...] = (jnp.mean(cap, keepdims=True) - 0.6) ** 2


def _run_router(xf, Wr, Wc, bc):
    scalar = jax.ShapeDtypeStruct((1, 1), jnp.float32)
    return pl.pallas_call(
        _router_body,
        in_specs=[
            pl.BlockSpec((S, H), lambda: (0, 0)),
            pl.BlockSpec((E, H), lambda: (0, 0)),
            pl.BlockSpec((1, H), lambda: (0, 0)),
            pl.BlockSpec(memory_space=pltpu.SMEM),
        ],
        out_shape=[
            jax.ShapeDtypeStruct((S, 2), jnp.int32),
            jax.ShapeDtypeStruct((S, 2), jnp.float32),
            jax.ShapeDtypeStruct((NP, S, WP), jnp.float32),
            scalar, scalar, scalar, scalar,
        ],
    )(xf, Wr, Wc, bc)


def _ffn_body(xg_ref, wg_ref, wu_ref, wd_ref, yg_ref):
    x0 = xg_ref[0]
    x1 = xg_ref[1]
    wg = wg_ref[0]
    wu = wu_ref[0]
    wd = wd_ref[0]
    dims = (((1,), (1,)), ((), ()))
    g = (jax.lax.dot_general(x0, wg[:, :WP], dims,
                             preferred_element_type=jnp.float32)
         + jax.lax.dot_general(x1, wg[:, WP:], dims,
                               preferred_element_type=jnp.float32))
    u = (jax.lax.dot_general(x0, wu[:, :WP], dims,
                             preferred_element_type=jnp.float32)
         + jax.lax.dot_general(x1, wu[:, WP:], dims,
                               preferred_element_type=jnp.float32))
    h = g * jax.nn.sigmoid(g) * u
    part = jax.lax.dot_general(h, wd, dims,
                               preferred_element_type=jnp.float32)

    @pl.when(pl.program_id(1) == 0)
    def _init():
        yg_ref[0] = part[:, :WP]
        yg_ref[1] = part[:, WP:]

    @pl.when(pl.program_id(1) != 0)
    def _acc():
        yg_ref[0] += part[:, :WP]
        yg_ref[1] += part[:, WP:]


def _run_ffn(xg_p, Wg, Wu, Wd):
    return pl.pallas_call(
        _ffn_body,
        grid=(E, NI),
        in_specs=[
            pl.BlockSpec((NP, C, WP), lambda e, i: (0, e, 0)),
            pl.BlockSpec((1, IBLK, H), lambda e, i: (e, i, 0)),
            pl.BlockSpec((1, IBLK, H), lambda e, i: (e, i, 0)),
            pl.BlockSpec((1, H, IBLK), lambda e, i: (e, 0, i)),
        ],
        out_specs=pl.BlockSpec((NP, C, WP), lambda e, i: (0, e, 0)),
        out_shape=jax.ShapeDtypeStruct((NP, XG_ROWS, WP), jnp.float32),
    )(xg_p, Wg, Wu, Wd)


def _vec_mesh():
    return plsc.VectorSubcoreMesh(core_axis_name="core",
                                  subcore_axis_name="subcore")


@jax.jit
def _sc_dispatch(x_p, sp):
    @pl.kernel(out_type=jax.ShapeDtypeStruct((NP * XG_ROWS, WP), jnp.float32),
               mesh=_vec_mesh())
    def k(x_hbm, s_hbm, o_hbm):
        def body(x_vmem, s_vmem):
            pltpu.sync_copy(x_vmem, o_hbm.at[s_vmem.at[0]])

        nxb = NP * S // IW
        pltpu.emit_pipeline(
            body,
            grid=(NIDX // IW,),
            in_specs=[
                pl.BlockSpec((IW, WP), index_map=lambda i: (i % nxb, 0)),
                pl.BlockSpec((1, IW), index_map=lambda i: (0, i)),
            ],
            out_specs=[],
            core_axis_name=("core", "subcore"),
            dimension_semantics=(pltpu.PARALLEL,),
        )(x_hbm, s_hbm)

    return k(x_p, sp)


@jax.jit
def _sc_gather(yg_p, sp):
    @pl.kernel(out_type=jax.ShapeDtypeStruct((NIDX, WP), jnp.float32),
               mesh=_vec_mesh())
    def k(y_hbm, s_hbm, o_hbm):
        def body(s_vmem, o_vmem):
            pltpu.sync_copy(y_hbm.at[s_vmem.at[0]], o_vmem)

        pltpu.emit_pipeline(
            body,
            grid=(NIDX // IW,),
            in_specs=[pl.BlockSpec((1, IW), index_map=lambda i: (0, i))],
            out_specs=[pl.BlockSpec((IW, WP), index_map=lambda i: (i, 0))],
            core_axis_name=("core", "subcore"),
            dimension_semantics=(pltpu.PARALLEL,),
        )(s_hbm, o_hbm)

    return k(yg_p, sp)


def _combine_body(g1_ref, g2_ref, w_ref, o_ref):
    w = w_ref[...]
    w1 = w[:, 0:1]
    w2 = w[:, 1:2]
    o_ref[...] = jnp.concatenate(
        [w1 * g1_ref[0, 0] + w2 * g2_ref[0, 0],
         w1 * g1_ref[0, 1] + w2 * g2_ref[0, 1]], axis=1)


def _run_combine(gath4, wts):
    return pl.pallas_call(
        _combine_body,
        grid=(S // TS,),
        in_specs=[
            pl.BlockSpec((1, NP, TS, WP), lambda i: (0, 0, i, 0)),
            pl.BlockSpec((1, NP, TS, WP), lambda i: (1, 0, i, 0)),
            pl.BlockSpec((TS, 2), lambda i: (i, 0)),
        ],
        out_specs=pl.BlockSpec((TS, H), lambda i: (i, 0)),
        out_shape=jax.ShapeDtypeStruct((S, H), jnp.float32),
    )(gath4, gath4, wts)


def kernel(x, Wr, Wc, bc, Wg, Wu, Wd):
    B_, S_, H_ = x.shape
    xf = x.reshape(S, H)
    slots, wts, x_p, lbl, z, div, closs = _run_router(xf, Wr, Wc, bc)
    plane_off = jnp.arange(NP, dtype=jnp.int32)[None, :, None] * XG_ROWS
    s_kt = jnp.transpose(slots)
    sp = (plane_off + s_kt[:, None, :]).reshape(1, NIDX)
    xg_p = _sc_dispatch(x_p.reshape(NP * S, WP), sp)
    yg_p = _run_ffn(xg_p.reshape(NP, XG_ROWS, WP), Wg, Wu, Wd)
    gath = _sc_gather(yg_p.reshape(NP * XG_ROWS, WP), sp)
    out = _run_combine(gath.reshape(K, NP, S, WP), wts)
    return (out.reshape(B_, S_, H_),
            lbl.reshape(()), z.reshape(()), div.reshape(()), closs.reshape(()))


# speedup vs baseline: 17.4670x; 1.0710x over previous
import jax
import jax.numpy as jnp
from jax.experimental import pallas as pl
from jax.experimental.pallas import tpu as pltpu
from jax.experimental.pallas import tpu_sc as plsc

E = 8
K = 2
S = 2048
H = 768
I = 3072
C = 768
EC = E * C
XG_ROWS = EC + 8
IBLK = 1536
NI = I // IBLK
TS = 256

NP = 2
WP = H // NP
NIDX = K * NP * S
IW = 128

_HIGH = jax.lax.Precision.HIGHEST


def _router_body(x_ref, wr_ref, wc_ref, bc_ref,
                 slots_ref, w_ref, xp_ref, lbl_ref, z_ref, div_ref, cap_ref):
    x = x_ref[...]
    xp_ref[0] = x[:, :WP]
    xp_ref[1] = x[:, WP:]
    logits = jax.lax.dot_general(x, wr_ref[...], (((1,), (1,)), ((), ())))
    m = jnp.max(logits, axis=1, keepdims=True)
    ex = jnp.exp(logits - m)
    probs = ex / jnp.sum(ex, axis=1, keepdims=True)
    cap = jax.nn.sigmoid(
        jnp.sum(x * wc_ref[...], axis=1, keepdims=True) + bc_ref[0])

    idx = jax.lax.broadcasted_iota(jnp.int32, (S, E), 1)
    m1 = jnp.max(probs, axis=1, keepdims=True)
    i1 = jnp.min(jnp.where(probs == m1, idx, E), axis=1, keepdims=True)
    one1 = idx == i1
    masked = jnp.where(one1, -jnp.inf, probs)
    m2 = jnp.max(masked, axis=1, keepdims=True)
    i2 = jnp.min(jnp.where(masked == m2, idx, E), axis=1, keepdims=True)
    one2 = idx == i2

    t = jnp.exp(m2 - m1)
    w1 = cap / (1.0 + t)
    w2 = cap * t / (1.0 + t)

    cnt = one1.astype(jnp.float32) + one2.astype(jnp.float32)
    ir = jax.lax.broadcasted_iota(jnp.int32, (S, S), 0)
    ic = jax.lax.broadcasted_iota(jnp.int32, (S, S), 1)
    tri = (ir > ic).astype(jnp.float32)
    r_excl = jax.lax.dot_general(tri, cnt, (((1,), (0,)), ((), ())),
                                 precision=_HIGH)
    r1 = jnp.sum(jnp.where(one1, r_excl, 0.0), axis=1, keepdims=True)
    r2 = jnp.sum(jnp.where(one2, r_excl, 0.0), axis=1, keepdims=True)
    r1 = r1.astype(jnp.int32)
    r2 = r2.astype(jnp.int32)
    ok1 = r1 < C
    ok2 = r2 < C
    s1 = jnp.where(ok1, i1 * C + r1, EC)
    s2 = jnp.where(ok2, i2 * C + r2, EC)
    w1 = jnp.where(ok1, w1, 0.0)
    w2 = jnp.where(ok2, w2, 0.0)

    slots_ref[...] = jnp.concatenate([s1, s2], axis=1)
    w_ref[...] = jnp.concatenate([w1, w2], axis=1)

    counts = jnp.sum(cnt, axis=0, keepdims=True)
    mean_load = float(S * K) / E
    lbl_ref[...] = (jnp.sum((counts - mean_load) ** 2, axis=1, keepdims=True)
                    / float(E - 1) / (mean_load * mean_load))
    z_ref[...] = jnp.mean(
        jnp.log(jnp.sum(jnp.exp(probs), axis=1, keepdims=True)),
        keepdims=True)
    ep = jnp.mean(probs, axis=0, keepdims=True)
    div_ref[...] = -jnp.sum(ep * jnp.log(ep + 1e-8), axis=1, keepdims=True)
    cap_ref[...] = (jnp.mean(cap, keepdims=True) - 0.6) ** 2


def _run_router(xf, Wr, Wc, bc):
    scalar = jax.ShapeDtypeStruct((1, 1), jnp.float32)
    return pl.pallas_call(
        _router_body,
        in_specs=[
            pl.BlockSpec((S, H), lambda: (0, 0)),
            pl.BlockSpec((E, H), lambda: (0, 0)),
            pl.BlockSpec((1, H), lambda: (0, 0)),
            pl.BlockSpec(memory_space=pltpu.SMEM),
        ],
        out_shape=[
            jax.ShapeDtypeStruct((S, 2), jnp.int32),
            jax.ShapeDtypeStruct((S, 2), jnp.float32),
            jax.ShapeDtypeStruct((NP, S, WP), jnp.float32),
            scalar, scalar, scalar, scalar,
        ],
    )(xf, Wr, Wc, bc)


def _ffn_body(xg_ref, wg_ref, wu_ref, wd_ref, yg_ref):
    x0 = xg_ref[0]
    x1 = xg_ref[1]
    wg = wg_ref[0]
    wu = wu_ref[0]
    wd = wd_ref[0]
    dims = (((1,), (1,)), ((), ()))
    g = (jax.lax.dot_general(x0, wg[:, :WP], dims,
                             preferred_element_type=jnp.float32)
         + jax.lax.dot_general(x1, wg[:, WP:], dims,
                               preferred_element_type=jnp.float32))
    u = (jax.lax.dot_general(x0, wu[:, :WP], dims,
                             preferred_element_type=jnp.float32)
         + jax.lax.dot_general(x1, wu[:, WP:], dims,
                               preferred_element_type=jnp.float32))
    h = g * jax.nn.sigmoid(g) * u
    part = jax.lax.dot_general(h, wd, dims,
                               preferred_element_type=jnp.float32)

    @pl.when(pl.program_id(1) == 0)
    def _init():
        yg_ref[0] = part[:, :WP]
        yg_ref[1] = part[:, WP:]

    @pl.when(pl.program_id(1) != 0)
    def _acc():
        yg_ref[0] += part[:, :WP]
        yg_ref[1] += part[:, WP:]


def _run_ffn(xg_p, Wg, Wu, Wd):
    return pl.pallas_call(
        _ffn_body,
        grid=(E, NI),
        in_specs=[
            pl.BlockSpec((NP, C, WP), lambda e, i: (0, e, 0)),
            pl.BlockSpec((1, IBLK, H), lambda e, i: (e, i, 0)),
            pl.BlockSpec((1, IBLK, H), lambda e, i: (e, i, 0)),
            pl.BlockSpec((1, H, IBLK), lambda e, i: (e, 0, i)),
        ],
        out_specs=pl.BlockSpec((NP, C, WP), lambda e, i: (0, e, 0)),
        out_shape=jax.ShapeDtypeStruct((NP, XG_ROWS, WP), jnp.float32),
    )(xg_p, Wg, Wu, Wd)


def _vec_mesh():
    return plsc.VectorSubcoreMesh(core_axis_name="core",
                                  subcore_axis_name="subcore")


@jax.jit
def _sc_dispatch(x_p, sp):
    @pl.kernel(out_type=jax.ShapeDtypeStruct((NP * XG_ROWS, WP), jnp.float32),
               mesh=_vec_mesh())
    def k(x_hbm, s_hbm, o_hbm):
        def body(x_vmem, s_vmem):
            pltpu.sync_copy(x_vmem, o_hbm.at[s_vmem.at[0]])

        nxb = NP * S // IW
        pltpu.emit_pipeline(
            body,
            grid=(NIDX // IW,),
            in_specs=[
                pl.BlockSpec((IW, WP), index_map=lambda i: (i % nxb, 0)),
                pl.BlockSpec((1, IW), index_map=lambda i: (0, i)),
            ],
            out_specs=[],
            core_axis_name=("core", "subcore"),
            dimension_semantics=(pltpu.PARALLEL,),
        )(x_hbm, s_hbm)

    return k(x_p, sp)


@jax.jit
def _sc_gather(yg_p, sp):
    @pl.kernel(out_type=jax.ShapeDtypeStruct((NIDX, WP), jnp.float32),
               mesh=_vec_mesh())
    def k(y_hbm, s_hbm, o_hbm):
        def body(s_vmem, o_vmem):
            pltpu.sync_copy(y_hbm.at[s_vmem.at[0]], o_vmem)

        pltpu.emit_pipeline(
            body,
            grid=(NIDX // IW,),
            in_specs=[pl.BlockSpec((1, IW), index_map=lambda i: (0, i))],
            out_specs=[pl.BlockSpec((IW, WP), index_map=lambda i: (i, 0))],
            core_axis_name=("core", "subcore"),
            dimension_semantics=(pltpu.PARALLEL,),
        )(s_hbm, o_hbm)

    return k(yg_p, sp)


def _combine_body(g1_ref, g2_ref, w_ref, o_ref):
    w = w_ref[...]
    w1 = w[:, 0:1]
    w2 = w[:, 1:2]
    o_ref[...] = jnp.concatenate(
        [w1 * g1_ref[0, 0] + w2 * g2_ref[0, 0],
         w1 * g1_ref[0, 1] + w2 * g2_ref[0, 1]], axis=1)


def _run_combine(gath4, wts):
    return pl.pallas_call(
        _combine_body,
        grid=(S // TS,),
        in_specs=[
            pl.BlockSpec((1, NP, TS, WP), lambda i: (0, 0, i, 0)),
            pl.BlockSpec((1, NP, TS, WP), lambda i: (1, 0, i, 0)),
            pl.BlockSpec((TS, 2), lambda i: (i, 0)),
        ],
        out_specs=pl.BlockSpec((TS, H), lambda i: (i, 0)),
        out_shape=jax.ShapeDtypeStruct((S, H), jnp.float32),
    )(gath4, gath4, wts)


def kernel(x, Wr, Wc, bc, Wg, Wu, Wd):
    B_, S_, H_ = x.shape
    xf = x.reshape(S, H)
    slots, wts, x_p, lbl, z, div, closs = _run_router(xf, Wr, Wc, bc)
    plane_off = jnp.arange(NP, dtype=jnp.int32)[None, :, None] * XG_ROWS
    s_kt = jnp.transpose(slots)
    sp = (plane_off + s_kt[:, None, :]).reshape(1, NIDX)
    xg_p = _sc_dispatch(x_p.reshape(NP * S, WP), sp)
    yg_p = _run_ffn(xg_p.reshape(NP, XG_ROWS, WP), Wg, Wu, Wd)
    gath = _sc_gather(yg_p.reshape(NP * XG_ROWS, WP), sp)
    out = _run_combine(gath.reshape(K, NP, S, WP), wts)
    return (out.reshape(B_, S_, H_),
            lbl.reshape(()), z.reshape(()), div.reshape(()), closs.reshape(()))


# speedup vs baseline: 18.4042x; 1.0537x over previous
import jax
import jax.numpy as jnp
from jax.experimental import pallas as pl
from jax.experimental.pallas import tpu as pltpu
from jax.experimental.pallas import tpu_sc as plsc

E = 8
K = 2
S = 2048
H = 768
I = 3072
C = 768
EC = E * C
XG_ROWS = EC + 8
IBLK = 1536
NI = I // IBLK
TS = 256

NP = 2
WP = H // NP
NIDX = K * NP * S
IW = 128

_HIGH = jax.lax.Precision.HIGHEST


def _router_body(x_ref, wr_ref, wc_ref, bc_ref,
                 slots_ref, w_ref, xp_ref, lbl_ref, z_ref, div_ref, cap_ref):
    x = x_ref[...]
    xp_ref[0] = x[:, :WP]
    xp_ref[1] = x[:, WP:]
    logits = jax.lax.dot_general(x, wr_ref[...], (((1,), (1,)), ((), ())))
    m = jnp.max(logits, axis=1, keepdims=True)
    ex = jnp.exp(logits - m)
    probs = ex / jnp.sum(ex, axis=1, keepdims=True)
    cap = jax.nn.sigmoid(
        jnp.sum(x * wc_ref[...], axis=1, keepdims=True) + bc_ref[0])

    idx = jax.lax.broadcasted_iota(jnp.int32, (S, E), 1)
    m1 = jnp.max(probs, axis=1, keepdims=True)
    i1 = jnp.min(jnp.where(probs == m1, idx, E), axis=1, keepdims=True)
    one1 = idx == i1
    masked = jnp.where(one1, -jnp.inf, probs)
    m2 = jnp.max(masked, axis=1, keepdims=True)
    i2 = jnp.min(jnp.where(masked == m2, idx, E), axis=1, keepdims=True)
    one2 = idx == i2

    t = jnp.exp(m2 - m1)
    w1 = cap / (1.0 + t)
    w2 = cap * t / (1.0 + t)

    cnt = one1.astype(jnp.float32) + one2.astype(jnp.float32)
    ir = jax.lax.broadcasted_iota(jnp.int32, (S, S), 0)
    ic = jax.lax.broadcasted_iota(jnp.int32, (S, S), 1)
    tri = (ir > ic).astype(jnp.float32)
    r_excl = jax.lax.dot_general(tri, cnt, (((1,), (0,)), ((), ())))
    r1 = jnp.sum(jnp.where(one1, r_excl, 0.0), axis=1, keepdims=True)
    r2 = jnp.sum(jnp.where(one2, r_excl, 0.0), axis=1, keepdims=True)
    r1 = r1.astype(jnp.int32)
    r2 = r2.astype(jnp.int32)
    ok1 = r1 < C
    ok2 = r2 < C
    s1 = jnp.where(ok1, i1 * C + r1, EC)
    s2 = jnp.where(ok2, i2 * C + r2, EC)
    w1 = jnp.where(ok1, w1, 0.0)
    w2 = jnp.where(ok2, w2, 0.0)

    slots_ref[...] = jnp.concatenate([s1, s2], axis=1)
    w_ref[...] = jnp.concatenate([w1, w2], axis=1)

    counts = jnp.sum(cnt, axis=0, keepdims=True)
    mean_load = float(S * K) / E
    lbl_ref[...] = (jnp.sum((counts - mean_load) ** 2, axis=1, keepdims=True)
                    / float(E - 1) / (mean_load * mean_load))
    z_ref[...] = jnp.mean(
        jnp.log(jnp.sum(jnp.exp(probs), axis=1, keepdims=True)),
        keepdims=True)
    ep = jnp.mean(probs, axis=0, keepdims=True)
    div_ref[...] = -jnp.sum(ep * jnp.log(ep + 1e-8), axis=1, keepdims=True)
    cap_ref[...] = (jnp.mean(cap, keepdims=True) - 0.6) ** 2


def _run_router(xf, Wr, Wc, bc):
    scalar = jax.ShapeDtypeStruct((1, 1), jnp.float32)
    return pl.pallas_call(
        _router_body,
        in_specs=[
            pl.BlockSpec((S, H), lambda: (0, 0)),
            pl.BlockSpec((E, H), lambda: (0, 0)),
            pl.BlockSpec((1, H), lambda: (0, 0)),
            pl.BlockSpec(memory_space=pltpu.SMEM),
        ],
        out_shape=[
            jax.ShapeDtypeStruct((S, 2), jnp.int32),
            jax.ShapeDtypeStruct((S, 2), jnp.float32),
            jax.ShapeDtypeStruct((NP, S, WP), jnp.float32),
            scalar, scalar, scalar, scalar,
        ],
    )(xf, Wr, Wc, bc)


def _ffn_body(xg_ref, wg_ref, wu_ref, wd_ref, yg_ref):
    x0 = xg_ref[0]
    x1 = xg_ref[1]
    wg = wg_ref[0]
    wu = wu_ref[0]
    wd = wd_ref[0]
    dims = (((1,), (1,)), ((), ()))
    g = (jax.lax.dot_general(x0, wg[:, :WP], dims,
                             preferred_element_type=jnp.float32)
         + jax.lax.dot_general(x1, wg[:, WP:], dims,
                               preferred_element_type=jnp.float32))
    u = (jax.lax.dot_general(x0, wu[:, :WP], dims,
                             preferred_element_type=jnp.float32)
         + jax.lax.dot_general(x1, wu[:, WP:], dims,
                               preferred_element_type=jnp.float32))
    h = g * jax.nn.sigmoid(g) * u
    part = jax.lax.dot_general(h, wd, dims,
                               preferred_element_type=jnp.float32)

    @pl.when(pl.program_id(1) == 0)
    def _init():
        yg_ref[0] = part[:, :WP]
        yg_ref[1] = part[:, WP:]

    @pl.when(pl.program_id(1) != 0)
    def _acc():
        yg_ref[0] += part[:, :WP]
        yg_ref[1] += part[:, WP:]


def _run_ffn(xg_p, Wg, Wu, Wd):
    return pl.pallas_call(
        _ffn_body,
        grid=(E, NI),
        in_specs=[
            pl.BlockSpec((NP, C, WP), lambda e, i: (0, e, 0)),
            pl.BlockSpec((1, IBLK, H), lambda e, i: (e, i, 0)),
            pl.BlockSpec((1, IBLK, H), lambda e, i: (e, i, 0)),
            pl.BlockSpec((1, H, IBLK), lambda e, i: (e, 0, i)),
        ],
        out_specs=pl.BlockSpec((NP, C, WP), lambda e, i: (0, e, 0)),
        out_shape=jax.ShapeDtypeStruct((NP, XG_ROWS, WP), jnp.float32),
    )(xg_p, Wg, Wu, Wd)


def _vec_mesh():
    return plsc.VectorSubcoreMesh(core_axis_name="core",
                                  subcore_axis_name="subcore")


@jax.jit
def _sc_dispatch(x_p, sp):
    @pl.kernel(out_type=jax.ShapeDtypeStruct((NP * XG_ROWS, WP), jnp.float32),
               mesh=_vec_mesh())
    def k(x_hbm, s_hbm, o_hbm):
        def body(x_vmem, s_vmem):
            pltpu.sync_copy(x_vmem, o_hbm.at[s_vmem.at[0]])

        nxb = NP * S // IW
        pltpu.emit_pipeline(
            body,
            grid=(NIDX // IW,),
            in_specs=[
                pl.BlockSpec((IW, WP), index_map=lambda i: (i % nxb, 0)),
                pl.BlockSpec((1, IW), index_map=lambda i: (0, i)),
            ],
            out_specs=[],
            core_axis_name=("core", "subcore"),
            dimension_semantics=(pltpu.PARALLEL,),
        )(x_hbm, s_hbm)

    return k(x_p, sp)


@jax.jit
def _sc_gather(yg_p, sp):
    @pl.kernel(out_type=jax.ShapeDtypeStruct((NIDX, WP), jnp.float32),
               mesh=_vec_mesh())
    def k(y_hbm, s_hbm, o_hbm):
        def body(s_vmem, o_vmem):
            pltpu.sync_copy(y_hbm.at[s_vmem.at[0]], o_vmem)

        pltpu.emit_pipeline(
            body,
            grid=(NIDX // IW,),
            in_specs=[pl.BlockSpec((1, IW), index_map=lambda i: (0, i))],
            out_specs=[pl.BlockSpec((IW, WP), index_map=lambda i: (i, 0))],
            core_axis_name=("core", "subcore"),
            dimension_semantics=(pltpu.PARALLEL,),
        )(s_hbm, o_hbm)

    return k(yg_p, sp)


def _combine_body(g1_ref, g2_ref, w_ref, o_ref):
    w = w_ref[...]
    w1 = w[:, 0:1]
    w2 = w[:, 1:2]
    o_ref[...] = jnp.concatenate(
        [w1 * g1_ref[0, 0] + w2 * g2_ref[0, 0],
         w1 * g1_ref[0, 1] + w2 * g2_ref[0, 1]], axis=1)


def _run_combine(gath4, wts):
    return pl.pallas_call(
        _combine_body,
        grid=(S // TS,),
        in_specs=[
            pl.BlockSpec((1, NP, TS, WP), lambda i: (0, 0, i, 0)),
            pl.BlockSpec((1, NP, TS, WP), lambda i: (1, 0, i, 0)),
            pl.BlockSpec((TS, 2), lambda i: (i, 0)),
        ],
        out_specs=pl.BlockSpec((TS, H), lambda i: (i, 0)),
        out_shape=jax.ShapeDtypeStruct((S, H), jnp.float32),
    )(gath4, gath4, wts)


def kernel(x, Wr, Wc, bc, Wg, Wu, Wd):
    B_, S_, H_ = x.shape
    xf = x.reshape(S, H)
    slots, wts, x_p, lbl, z, div, closs = _run_router(xf, Wr, Wc, bc)
    plane_off = jnp.arange(NP, dtype=jnp.int32)[None, :, None] * XG_ROWS
    s_kt = jnp.transpose(slots)
    sp = (plane_off + s_kt[:, None, :]).reshape(1, NIDX)
    xg_p = _sc_dispatch(x_p.reshape(NP * S, WP), sp)
    yg_p = _run_ffn(xg_p.reshape(NP, XG_ROWS, WP), Wg, Wu, Wd)
    gath = _sc_gather(yg_p.reshape(NP * XG_ROWS, WP), sp)
    out = _run_combine(gath.reshape(K, NP, S, WP), wts)
    return (out.reshape(B_, S_, H_),
            lbl.reshape(()), z.reshape(()), div.reshape(()), closs.reshape(()))
